# baseline (speedup 1.0000x reference)
import jax
import jax.numpy as jnp
from jax import lax
from jax.experimental import pallas as pl
from jax.experimental.pallas import tpu as pltpu
from jax.experimental.pallas import tpu_sc as plsc

NUM_YEARS = 1000
LATENT_DIM = 64
BATCH = 16384

NC = 2
NS = 16
NW = NC * NS
B_PER_W = BATCH // NW


def _gather_kernel(idx_hbm, emb_hbm, out_hbm, idx_v, rows_v, table_sp, sem_g, sem_o):
    sid = lax.axis_index("s")
    wid = sid * NC + lax.axis_index("c")
    base = wid * B_PER_W

    rows_per_tile = NUM_YEARS // NS
    rem = NUM_YEARS - rows_per_tile * NS
    stage = pltpu.async_copy(
        emb_hbm.at[pl.ds(sid * rows_per_tile, rows_per_tile)],
        table_sp.at[pl.ds(sid * rows_per_tile, rows_per_tile)],
        sem_g,
    )
    @pl.when(sid == 0)
    def _stage_tail():
        pltpu.sync_copy(
            emb_hbm.at[pl.ds(NS * rows_per_tile, rem)],
            table_sp.at[pl.ds(NS * rows_per_tile, rem)],
        )

    pltpu.sync_copy(idx_hbm.at[pl.ds(base, B_PER_W)], idx_v)
    stage.wait()
    plsc.subcore_barrier()
    nchunk = 8
    step = B_PER_W // nchunk
    gathers = [
        pltpu.async_copy(
            table_sp.at[idx_v.at[pl.ds(c * step, step)]],
            rows_v.at[pl.ds(c * step, step)],
            sem_g,
        )
        for c in range(nchunk)
    ]
    writes = []
    for c in range(nchunk):
        gathers[c].wait()
        writes.append(
            pltpu.async_copy(
                rows_v.at[pl.ds(c * step, step)],
                out_hbm.at[pl.ds(base + c * step, step)],
                sem_o,
            )
        )
    for w in writes:
        w.wait()


@jax.jit
def kernel(session_year, emb):
    mesh = plsc.VectorSubcoreMesh(core_axis_name="c", subcore_axis_name="s")
    return pl.kernel(
        _gather_kernel,
        out_type=jax.ShapeDtypeStruct((BATCH, LATENT_DIM), jnp.float32),
        mesh=mesh,
        scratch_types=[
            pltpu.VMEM((B_PER_W,), jnp.int32),
            pltpu.VMEM((B_PER_W, LATENT_DIM), jnp.float32),
            pltpu.VMEM_SHARED((NUM_YEARS, LATENT_DIM), jnp.float32),
            pltpu.SemaphoreType.DMA,
            pltpu.SemaphoreType.DMA,
        ],
        compiler_params=pltpu.CompilerParams(use_tc_tiling_on_sc=False),
    )(session_year, emb)


# speedup vs baseline: 1.0106x; 1.0106x over previous
import jax
import jax.numpy as jnp
from jax import lax
from jax.experimental import pallas as pl
from jax.experimental.pallas import tpu as pltpu
from jax.experimental.pallas import tpu_sc as plsc

NUM_YEARS = 1000
LATENT_DIM = 64
BATCH = 16384

NC = 2
NS = 16
NW = NC * NS
B_PER_W = BATCH // NW


def _gather_kernel(idx_hbm, emb_hbm, out_hbm, idx_v, rows_v, table_sp, sem_g, sem_o):
    sid = lax.axis_index("s")
    wid = sid * NC + lax.axis_index("c")
    base = wid * B_PER_W

    rows_per_tile = NUM_YEARS // NS
    rem = NUM_YEARS - rows_per_tile * NS
    stage = pltpu.async_copy(
        emb_hbm.at[pl.ds(sid * rows_per_tile, rows_per_tile)],
        table_sp.at[pl.ds(sid * rows_per_tile, rows_per_tile)],
        sem_g,
    )
    @pl.when(sid == 0)
    def _stage_tail():
        pltpu.sync_copy(
            emb_hbm.at[pl.ds(NS * rows_per_tile, rem)],
            table_sp.at[pl.ds(NS * rows_per_tile, rem)],
        )

    pltpu.sync_copy(idx_hbm.at[pl.ds(base, B_PER_W)], idx_v)
    stage.wait()
    plsc.subcore_barrier()
    nchunk = 4
    step = B_PER_W // nchunk
    gathers = [
        pltpu.async_copy(
            table_sp.at[idx_v.at[pl.ds(c * step, step)]],
            rows_v.at[pl.ds(c * step, step)],
            sem_g,
        )
        for c in range(nchunk)
    ]
    writes = []
    for c in range(nchunk):
        gathers[c].wait()
        writes.append(
            pltpu.async_copy(
                rows_v.at[pl.ds(c * step, step)],
                out_hbm.at[pl.ds(base + c * step, step)],
                sem_o,
            )
        )
    for w in writes:
        w.wait()


@jax.jit
def kernel(session_year, emb):
    mesh = plsc.VectorSubcoreMesh(core_axis_name="c", subcore_axis_name="s")
    return pl.kernel(
        _gather_kernel,
        out_type=jax.ShapeDtypeStruct((BATCH, LATENT_DIM), jnp.float32),
        mesh=mesh,
        scratch_types=[
            pltpu.VMEM((B_PER_W,), jnp.int32),
            pltpu.VMEM((B_PER_W, LATENT_DIM), jnp.float32),
            pltpu.VMEM_SHARED((NUM_YEARS, LATENT_DIM), jnp.float32),
            pltpu.SemaphoreType.DMA,
            pltpu.SemaphoreType.DMA,
        ],
        compiler_params=pltpu.CompilerParams(use_tc_tiling_on_sc=False),
    )(session_year, emb)
